# baseline (speedup 1.0000x reference)
import functools
import math

import jax
import jax.numpy as jnp
from jax import lax
from jax.experimental import pallas as pl
from jax.experimental.pallas import tpu as pltpu
from jax.experimental.pallas import tpu_sc as plsc

EMBED = 128
OUT_D = 6 + 2 * EMBED
NC = 2
NS = 16
NW = NC * NS
LN = 128


def _transform_kernel(nr, ne, tr, te, wr, br, gr, betar, we, be, ge, betae,
                      outr, oute):
    inv_sqrt2 = 0.7071067811865476

    def tfm(x, W, b, g, beta, nvalid):
        y = lax.dot_general(x, W, (((1,), (1,)), ((), ())),
                            preferred_element_type=jnp.float32)
        y = y + b
        mu = jnp.mean(y, axis=-1, keepdims=True)
        var = jnp.mean((y - mu) ** 2, axis=-1, keepdims=True)
        y = (y - mu) / jnp.sqrt(var + 1e-5) * g + beta
        y = y * 0.5 * (1.0 + lax.erf(y * inv_sqrt2))
        rows = lax.broadcasted_iota(jnp.int32, y.shape, 0)
        return jnp.where(rows < nvalid, y, 0.0)

    outr[...] = tfm(tr[...], wr[...], br[...], gr[...], betar[...], nr)
    oute[...] = tfm(te[...], we[...], be[...], ge[...], betae[...], ne)


def _transform_tables(tr, te, Wr, br, gr, betar, We, be, ge, betae):
    nr, ne = tr.shape[0], te.shape[0]
    nr_pad = 8 * math.ceil((nr + 1) / 8)
    ne_pad = 8 * math.ceil((ne + 1) / 8)
    tr_p = jnp.zeros((nr_pad, EMBED), jnp.float32).at[:nr].set(tr)
    te_p = jnp.zeros((ne_pad, EMBED), jnp.float32).at[:ne].set(te)
    r2 = lambda v: v.reshape(1, EMBED)
    outr, oute = pl.pallas_call(
        functools.partial(_transform_kernel, nr, ne),
        out_shape=(jax.ShapeDtypeStruct((nr_pad, EMBED), jnp.float32),
                   jax.ShapeDtypeStruct((ne_pad, EMBED), jnp.float32)),
    )(tr_p, te_p, Wr, r2(br), r2(gr), r2(betar), We, r2(be), r2(ge), r2(betae))
    return outr, oute


def _sc_body(n_chunks, zrow_r, zrow_e, stride,
             tabr_hbm, tabe_hbm, ridT, eidT, lens_hbm, bbT, feat3,
             tabr, tabe, idsb, bbb, obufA, obufB, lens_vm,
             s_t0, s_t1, s_i0, s_i1, s_j0, s_j1, s_b0, s_b1, s_o0, s_o1,
             s_l):
    s_i = (s_i0, s_i1)
    s_j = (s_j0, s_j1)
    s_b = (s_b0, s_b1)
    s_o = (s_o0, s_o1)
    obufs = (obufA, obufB)
    wid = lax.axis_index("s") * NC + lax.axis_index("c")
    lane0 = pl.multiple_of(wid * LN, LN)

    ct0 = pltpu.async_copy(tabr_hbm, tabr, s_t0)
    ct1 = pltpu.async_copy(tabe_hbm, tabe, s_t1)
    cl = pltpu.async_copy(lens_hbm.at[pl.ds(lane0, LN)], lens_vm, s_l)

    def in_dmas(n, b):
        pltpu.async_copy(ridT.at[n, pl.ds(lane0, LN)], idsb.at[b, 0], s_i[b])
        pltpu.async_copy(eidT.at[n, pl.ds(lane0, LN)], idsb.at[b, 1], s_j[b])
        pltpu.async_copy(bbT.at[n, :, pl.ds(lane0, LN)], bbb.at[b], s_b[b])

    in_dmas(0, 0)
    in_dmas(1, 1)
    ct0.wait()
    ct1.wait()
    cl.wait()

    hi_mask = jnp.int32(-65536)

    def outer(g, carry):
        for b in range(2):
            n = 2 * g + b
            pltpu.make_async_copy(ridT.at[0, pl.ds(0, LN)], idsb.at[b, 0],
                                  s_i[b]).wait()
            pltpu.make_async_copy(eidT.at[0, pl.ds(0, LN)], idsb.at[b, 1],
                                  s_j[b]).wait()
            pltpu.make_async_copy(bbT.at[0, :, pl.ds(0, LN)], bbb.at[b],
                                  s_b[b]).wait()

            @pl.when(n >= 2)
            def _wait_out():
                pltpu.make_async_copy(
                    obufs[b], feat3.at[0, :, pl.ds(0, LN)], s_o[b]).wait()

            def group(v, carry2):
                sl = pl.ds(v * 16, 16)
                lv = lens_vm[sl]
                m = n < lv
                mf = jnp.where(m, 1.0, 0.0)
                ridx = jnp.where(m, idsb[b, 0, sl], zrow_r) * stride
                eidx = jnp.where(m, idsb[b, 1, sl], zrow_e) * stride
                x1 = bbb[b, 0, sl]
                y1 = bbb[b, 1, sl]
                x2 = bbb[b, 2, sl]
                y2 = bbb[b, 3, sl]
                w = x2 - x1
                h = y2 - y1
                feats = (x1, y1, x2, y2, w * h, w / (h + 1e-6))
                for f in range(6):
                    obufs[b][f, sl] = feats[f] * mf
                for k in range(EMBED // 2):
                    pr = plsc.load_gather(tabr, [ridx + k])
                    pe = plsc.load_gather(tabe, [eidx + k])
                    obufs[b][6 + 2 * k, sl] = plsc.bitcast(
                        pr << 16, jnp.float32)
                    obufs[b][7 + 2 * k, sl] = plsc.bitcast(
                        pr & hi_mask, jnp.float32)
                    obufs[b][6 + EMBED + 2 * k, sl] = plsc.bitcast(
                        pe << 16, jnp.float32)
                    obufs[b][7 + EMBED + 2 * k, sl] = plsc.bitcast(
                        pe & hi_mask, jnp.float32)
                return carry2

            lax.fori_loop(0, LN // 16, group, 0)
            pltpu.async_copy(obufs[b], feat3.at[n, :, pl.ds(lane0, LN)],
                             s_o[b])

            @pl.when(n + 2 < n_chunks)
            def _prefetch():
                in_dmas(n + 2, b)
        return carry

    lax.fori_loop(0, n_chunks // 2, outer, 0)
    for b in range(2):
        pltpu.make_async_copy(obufs[b], feat3.at[0, :, pl.ds(0, LN)],
                              s_o[b]).wait()


def _mask_kernel(lens_ref, mask_ref):
    n_iota = lax.broadcasted_iota(jnp.int32, mask_ref.shape, 1)
    mask_ref[...] = (n_iota < lens_ref[...]).astype(jnp.float32)


def _mask(lengths, N, b_blk=512):
    B = lengths.shape[0]
    return pl.pallas_call(
        _mask_kernel,
        grid=(B // b_blk,),
        in_specs=[pl.BlockSpec((b_blk, 1), lambda i: (i, 0))],
        out_specs=pl.BlockSpec((b_blk, N), lambda i: (i, 0)),
        out_shape=jax.ShapeDtypeStruct((B, N), jnp.float32),
    )(lengths.reshape(B, 1))


def kernel(bboxes, region_ids, entity_ids, lengths, region_table, entity_table,
           Wr, br, gr, betar, We, be, ge, betae):
    B, N = region_ids.shape
    assert B == NW * LN and N % 2 == 0

    tabr, tabe = _transform_tables(region_table, entity_table,
                                   Wr, br, gr, betar, We, be, ge, betae)
    zrow_r = region_table.shape[0]
    zrow_e = entity_table.shape[0]

    stride = EMBED // 2 + 1

    def pack(t):
        tu = lax.bitcast_convert_type(t.astype(jnp.bfloat16), jnp.uint16)
        tu = tu.astype(jnp.uint32).reshape(t.shape[0], EMBED // 2, 2)
        p = (tu[:, :, 0] | (tu[:, :, 1] << 16)).astype(jnp.int32)
        return jnp.pad(p, ((0, 0), (0, 1))).reshape(-1)

    tabr_f = pack(tabr)
    tabe_f = pack(tabe)

    lens32 = lengths.astype(jnp.int32)
    ridT = region_ids.astype(jnp.int32).T
    eidT = entity_ids.astype(jnp.int32).T
    bbT = bboxes.transpose(1, 2, 0)

    mesh = plsc.VectorSubcoreMesh(core_axis_name="c", subcore_axis_name="s")
    feat3 = pl.kernel(
        functools.partial(_sc_body, N, zrow_r, zrow_e, stride),
        out_type=jax.ShapeDtypeStruct((N, OUT_D, B), jnp.float32),
        mesh=mesh,
        compiler_params=pltpu.CompilerParams(use_tc_tiling_on_sc=True,
                                             needs_layout_passes=False),
        scratch_types=(
            pltpu.VMEM(tabr_f.shape, jnp.int32),
            pltpu.VMEM(tabe_f.shape, jnp.int32),
            pltpu.VMEM((2, 2, LN), jnp.int32),
            pltpu.VMEM((2, 4, LN), jnp.float32),
            pltpu.VMEM((OUT_D, LN), jnp.float32),
            pltpu.VMEM((OUT_D, LN), jnp.float32),
            pltpu.VMEM((LN,), jnp.int32),
            pltpu.SemaphoreType.DMA,
            pltpu.SemaphoreType.DMA,
            pltpu.SemaphoreType.DMA,
            pltpu.SemaphoreType.DMA,
            pltpu.SemaphoreType.DMA,
            pltpu.SemaphoreType.DMA,
            pltpu.SemaphoreType.DMA,
            pltpu.SemaphoreType.DMA,
            pltpu.SemaphoreType.DMA,
            pltpu.SemaphoreType.DMA,
            pltpu.SemaphoreType.DMA,
        ),
    )(tabr_f, tabe_f, ridT, eidT, lens32, bbT)

    feat = feat3.transpose(2, 0, 1)
    mask = _mask(lens32, N)
    return feat, mask
